# baseline (speedup 1.0000x reference)
import functools

import jax
import jax.numpy as jnp
from jax import lax
from jax.experimental import pallas as pl
from jax.experimental.pallas import tpu as pltpu
from jax.experimental.pallas import tpu_sc as plsc


_CHUNK_ROWS = 16


def kernel(x):
    b, s, d = x.shape
    rows = b * s
    x2 = x.reshape(rows, d)
    info = plsc.get_sparse_core_info()
    nc, ns = info.num_cores, info.num_subcores
    nw = nc * ns
    rpw = rows // nw
    n_chunks = rpw // _CHUNK_ROWS
    mesh = plsc.VectorSubcoreMesh(core_axis_name="c", subcore_axis_name="s")

    @functools.partial(
        pl.kernel,
        out_type=jax.ShapeDtypeStruct((rows, d), x.dtype),
        mesh=mesh,
        scratch_types=[
            pltpu.VMEM((_CHUNK_ROWS, d), x.dtype),
            pltpu.VMEM((_CHUNK_ROWS, d), x.dtype),
            pltpu.SemaphoreType.DMA,
            pltpu.SemaphoreType.DMA,
            pltpu.SemaphoreType.DMA,
            pltpu.SemaphoreType.DMA,
        ],
    )
    def sc_copy(x_hbm, o_hbm, buf0, buf1, rsem0, rsem1, wsem0, wsem1):
        wid = lax.axis_index("s") * nc + lax.axis_index("c")
        base = wid * rpw
        bufs = (buf0, buf1)
        rsems = (rsem0, rsem1)
        wsems = (wsem0, wsem1)

        def rd(j):
            return pltpu.async_copy(
                x_hbm.at[pl.ds(base + j * _CHUNK_ROWS, _CHUNK_ROWS), :],
                bufs[j % 2],
                rsems[j % 2],
            )

        def wr(j):
            return pltpu.async_copy(
                bufs[j % 2],
                o_hbm.at[pl.ds(base + j * _CHUNK_ROWS, _CHUNK_ROWS), :],
                wsems[j % 2],
            )

        reads = {0: rd(0)}
        writes = {}
        for j in range(n_chunks):
            reads.pop(j).wait()
            writes[j] = wr(j)
            if j + 1 < n_chunks:
                if j >= 1:
                    writes.pop(j - 1).wait()
                reads[j + 1] = rd(j + 1)
        for h in writes.values():
            h.wait()

    return sc_copy(x2).reshape(b, s, d)
